# Initial kernel scaffold; baseline (speedup 1.0000x reference)
import jax
import jax.numpy as jnp
from jax.experimental import pallas as pl


def kernel(x, edge_index, Wl1, Wr1, b1, Wl2, Wr2, b2, Wl3, Wr3, b3):
    raise NotImplementedError("write your pallas kernel here")


# speedup vs baseline: 9.5256x; 9.5256x over previous
import functools

import jax
import jax.numpy as jnp
from jax import lax
from jax.experimental import pallas as pl
from jax.experimental.pallas import tpu as pltpu
from jax.experimental.pallas import tpu_sc as plsc

NC = 2
NS = 16
NW = NC * NS
CH = 128


def _make_sc_agg(n_pad, d, n_chunks, with_deg):
  rows_per_sub = n_pad // NS
  mesh = plsc.VectorSubcoreMesh(core_axis_name="c", subcore_axis_name="s")

  out_type = [jax.ShapeDtypeStruct((NC, n_pad, d), jnp.float32)]
  scratch = [
      pltpu.VMEM((n_chunks, CH), jnp.int32),
      pltpu.VMEM((n_chunks, CH), jnp.int32),
      pltpu.VMEM((CH, d), jnp.float32),
      pltpu.SemaphoreType.DMA,
      pltpu.VMEM_SHARED((n_pad, d), jnp.float32),
  ]
  if with_deg:
    out_type.append(jax.ShapeDtypeStruct((NC, n_pad, 16), jnp.float32))
    scratch += [
        pltpu.VMEM((CH, 16), jnp.float32),
        pltpu.VMEM_SHARED((n_pad, 16), jnp.float32),
    ]

  def body(*refs):
    if with_deg:
      (p_hbm, src_hbm, dst_hbm, z_hbm, z16_hbm, ones_hbm,
       agg_out, deg_out, src_v, dst_v, rows_v, sem, acc_sh,
       ones_v, deg_sh) = refs
    else:
      (p_hbm, src_hbm, dst_hbm, z_hbm,
       agg_out, src_v, dst_v, rows_v, sem, acc_sh) = refs

    c = lax.axis_index("c")
    s = lax.axis_index("s")
    w = c * NS + s
    lo = s * rows_per_sub
    rsl = pl.ds(lo, rows_per_sub)

    pltpu.sync_copy(z_hbm.at[rsl], acc_sh.at[rsl])
    if with_deg:
      pltpu.sync_copy(z16_hbm.at[rsl], deg_sh.at[rsl])
      pltpu.sync_copy(ones_hbm, ones_v)

    pltpu.sync_copy(src_hbm.at[w], src_v)
    pltpu.sync_copy(dst_hbm.at[w], dst_v)
    plsc.subcore_barrier()

    @pl.loop(0, n_chunks)
    def _(j):
      pltpu.async_copy(p_hbm.at[src_v.at[j]], rows_v, sem).wait()
      pltpu.sync_copy(rows_v, acc_sh.at[dst_v.at[j]], add=True)
      if with_deg:
        pltpu.sync_copy(ones_v, deg_sh.at[dst_v.at[j]], add=True)

    plsc.subcore_barrier()
    pltpu.sync_copy(acc_sh.at[rsl], agg_out.at[c, rsl])
    if with_deg:
      pltpu.sync_copy(deg_sh.at[rsl], deg_out.at[c, rsl])

  return pl.kernel(
      body, out_type=out_type, mesh=mesh, scratch_types=scratch,
      compiler_params=pltpu.CompilerParams(use_tc_tiling_on_sc=False))


def _proj2_call(x, wl, wr, blk):
  n, k = x.shape
  d = wl.shape[1]

  def body(x_ref, wl_ref, wr_ref, p_ref, r_ref):
    xb = x_ref[...]
    p_ref[...] = jnp.dot(xb, wl_ref[...], preferred_element_type=jnp.float32)
    r_ref[...] = jnp.dot(xb, wr_ref[...], preferred_element_type=jnp.float32)

  return pl.pallas_call(
      body,
      grid=(n // blk,),
      in_specs=[
          pl.BlockSpec((blk, k), lambda i: (i, 0)),
          pl.BlockSpec((k, d), lambda i: (0, 0)),
          pl.BlockSpec((k, d), lambda i: (0, 0)),
      ],
      out_specs=[
          pl.BlockSpec((blk, d), lambda i: (i, 0)),
          pl.BlockSpec((blk, d), lambda i: (i, 0)),
      ],
      out_shape=[
          jax.ShapeDtypeStruct((n, d), jnp.float32),
          jax.ShapeDtypeStruct((n, d), jnp.float32),
      ],
  )(x, wl, wr)


def _mid_layer_call(agg, degp, r, b, wl, wr, blk, first):
  _, n, d = agg.shape
  do = wl.shape[1]

  def body(a_ref, g_ref, r_ref, b_ref, wl_ref, wr_ref, *o_refs):
    a = a_ref[0] + a_ref[1]
    if first:
      deg = jnp.maximum(g_ref[0] + g_ref[1], 1.0)
      dinv = 1.0 / deg
    else:
      dinv = g_ref[...]
    h = jnp.maximum(a * dinv[:, 0:1] + r_ref[...] + b_ref[...], 0.0)
    o_refs[0][...] = jnp.dot(h, wl_ref[...], preferred_element_type=jnp.float32)
    o_refs[1][...] = jnp.dot(h, wr_ref[...], preferred_element_type=jnp.float32)
    if first:
      o_refs[2][...] = dinv

  g_spec = (pl.BlockSpec((NC, blk, 16), lambda i: (0, i, 0)) if first
            else pl.BlockSpec((blk, 16), lambda i: (i, 0)))
  out_specs = [pl.BlockSpec((blk, do), lambda i: (i, 0)),
               pl.BlockSpec((blk, do), lambda i: (i, 0))]
  out_shape = [jax.ShapeDtypeStruct((n, do), jnp.float32),
               jax.ShapeDtypeStruct((n, do), jnp.float32)]
  if first:
    out_specs.append(pl.BlockSpec((blk, 16), lambda i: (i, 0)))
    out_shape.append(jax.ShapeDtypeStruct((n, 16), jnp.float32))

  return pl.pallas_call(
      body,
      grid=(n // blk,),
      in_specs=[
          pl.BlockSpec((NC, blk, d), lambda i: (0, i, 0)),
          g_spec,
          pl.BlockSpec((blk, d), lambda i: (i, 0)),
          pl.BlockSpec((1, d), lambda i: (0, 0)),
          pl.BlockSpec((d, do), lambda i: (0, 0)),
          pl.BlockSpec((d, do), lambda i: (0, 0)),
      ],
      out_specs=out_specs,
      out_shape=out_shape,
  )(agg, degp, r, b, wl, wr)


def _final_call(agg, dinv, r, b, blk):
  _, n, d = agg.shape

  def body(a_ref, g_ref, r_ref, b_ref, o_ref):
    a = a_ref[0] + a_ref[1]
    o_ref[...] = a * g_ref[...][:, 0:1] + r_ref[...] + b_ref[...]

  return pl.pallas_call(
      body,
      grid=(n // blk,),
      in_specs=[
          pl.BlockSpec((NC, blk, d), lambda i: (0, i, 0)),
          pl.BlockSpec((blk, 16), lambda i: (i, 0)),
          pl.BlockSpec((blk, d), lambda i: (i, 0)),
          pl.BlockSpec((1, d), lambda i: (0, 0)),
      ],
      out_specs=pl.BlockSpec((blk, d), lambda i: (i, 0)),
      out_shape=jax.ShapeDtypeStruct((n, d), jnp.float32),
  )(agg, dinv, r, b)


def kernel(x, edge_index, Wl1, Wr1, b1, Wl2, Wr2, b2, Wl3, Wr3, b3):
  n, d_in = x.shape
  e = edge_index.shape[1]
  d_h = Wl1.shape[1]
  n_cls = Wl3.shape[1]
  do = 16

  n_pad = -(-(n + 1) // 128) * 128
  per_tile = -(-e // (NW * CH)) * CH
  n_chunks = per_tile // CH
  e_pad = per_tile * NW

  src = edge_index[0]
  dst = edge_index[1]
  pad = e_pad - e
  pad_src = (jnp.arange(pad, dtype=jnp.int32) * 97) % n
  pad_dst = n + (jnp.arange(pad, dtype=jnp.int32) % (n_pad - n))
  srcC = jnp.concatenate([src, pad_src]).reshape(NW, n_chunks, CH)
  dstC = jnp.concatenate([dst, pad_dst]).reshape(NW, n_chunks, CH)

  zeros_d = jnp.zeros((n_pad, d_h), jnp.float32)
  zeros_16 = jnp.zeros((n_pad, 16), jnp.float32)
  ones_16 = jnp.ones((CH, 16), jnp.float32)

  wl3p = jnp.zeros((d_h, do), jnp.float32).at[:, :n_cls].set(Wl3)
  wr3p = jnp.zeros((d_h, do), jnp.float32).at[:, :n_cls].set(Wr3)
  b3p = jnp.zeros((1, do), jnp.float32).at[0, :n_cls].set(b3)

  blk = 1000

  p1, r1 = _proj2_call(x, Wl1, Wr1, blk)
  agg1, degp = _make_sc_agg(n_pad, d_h, n_chunks, True)(
      p1, srcC, dstC, zeros_d, zeros_16, ones_16)
  p2, r2, dinv = _mid_layer_call(
      agg1[:, :n], degp[:, :n], r1, b1.reshape(1, d_h), Wl2, Wr2, blk, True)

  agg2, = _make_sc_agg(n_pad, d_h, n_chunks, False)(p2, srcC, dstC, zeros_d)
  p3, r3 = _mid_layer_call(
      agg2[:, :n], dinv, r2, b2.reshape(1, d_h), wl3p, wr3p, blk, False)

  zeros_do = zeros_16 if do == 16 else jnp.zeros((n_pad, do), jnp.float32)
  agg3, = _make_sc_agg(n_pad, do, n_chunks, False)(p3, srcC, dstC, zeros_do)
  out = _final_call(agg3[:, :n], dinv, r3, b3p, blk)
  return out[:, :n_cls]


# speedup vs baseline: 13.1243x; 1.3778x over previous
import functools

import jax
import jax.numpy as jnp
from jax import lax
from jax.experimental import pallas as pl
from jax.experimental.pallas import tpu as pltpu
from jax.experimental.pallas import tpu_sc as plsc

NC = 2
NS = 16
NW = NC * NS
CH = 128


def _make_sc_agg(n_pad, d, n_chunks, with_deg):
  rows_per_sub = n_pad // NS
  mesh = plsc.VectorSubcoreMesh(core_axis_name="c", subcore_axis_name="s")

  out_type = [jax.ShapeDtypeStruct((NC, n_pad, d), jnp.float32)]
  scratch = [
      pltpu.VMEM((n_chunks, CH), jnp.int32),
      pltpu.VMEM((n_chunks, CH), jnp.int32),
      pltpu.VMEM((CH, d), jnp.float32),
      pltpu.VMEM((CH, d), jnp.float32),
      pltpu.SemaphoreType.DMA,
      pltpu.SemaphoreType.DMA,
      pltpu.VMEM_SHARED((n_pad, d), jnp.float32),
  ]
  if with_deg:
    out_type.append(jax.ShapeDtypeStruct((NC, n_pad, 16), jnp.float32))
    scratch += [
        pltpu.VMEM((CH, 16), jnp.float32),
        pltpu.VMEM_SHARED((n_pad, 16), jnp.float32),
    ]

  def body(*refs):
    if with_deg:
      (p_hbm, src_hbm, dst_hbm, z_hbm, z16_hbm, ones_hbm,
       agg_out, deg_out, src_v, dst_v, rows_a, rows_b, sem_a, sem_b, acc_sh,
       ones_v, deg_sh) = refs
    else:
      (p_hbm, src_hbm, dst_hbm, z_hbm,
       agg_out, src_v, dst_v, rows_a, rows_b, sem_a, sem_b, acc_sh) = refs

    c = lax.axis_index("c")
    s = lax.axis_index("s")
    w = c * NS + s
    lo = s * rows_per_sub
    rsl = pl.ds(lo, rows_per_sub)

    pltpu.sync_copy(z_hbm.at[rsl], acc_sh.at[rsl])
    if with_deg:
      pltpu.sync_copy(z16_hbm.at[rsl], deg_sh.at[rsl])
      pltpu.sync_copy(ones_hbm, ones_v)

    pltpu.sync_copy(src_hbm.at[w], src_v)
    pltpu.sync_copy(dst_hbm.at[w], dst_v)
    plsc.subcore_barrier()

    def gather(j, buf, sem):
      pltpu.async_copy(p_hbm.at[src_v.at[j]], buf, sem)

    def drain_scatter(j, buf, sem):
      pltpu.make_async_copy(p_hbm.at[src_v.at[j]], buf, sem).wait()
      pltpu.sync_copy(buf, acc_sh.at[dst_v.at[j]], add=True)
      if with_deg:
        pltpu.sync_copy(ones_v, deg_sh.at[dst_v.at[j]], add=True)

    gather(0, rows_a, sem_a)

    @pl.loop(0, n_chunks, step=2)
    def _(j):
      gather(j + 1, rows_b, sem_b)
      drain_scatter(j, rows_a, sem_a)

      @pl.when(j + 2 < n_chunks)
      def _():
        gather(j + 2, rows_a, sem_a)

      drain_scatter(j + 1, rows_b, sem_b)

    plsc.subcore_barrier()
    pltpu.sync_copy(acc_sh.at[rsl], agg_out.at[c, rsl])
    if with_deg:
      pltpu.sync_copy(deg_sh.at[rsl], deg_out.at[c, rsl])

  return pl.kernel(
      body, out_type=out_type, mesh=mesh, scratch_types=scratch,
      compiler_params=pltpu.CompilerParams(use_tc_tiling_on_sc=False))


def _proj2_call(x, wl, wr, blk):
  n, k = x.shape
  d = wl.shape[1]

  def body(x_ref, wl_ref, wr_ref, p_ref, r_ref):
    xb = x_ref[...]
    p_ref[...] = jnp.dot(xb, wl_ref[...], preferred_element_type=jnp.float32)
    r_ref[...] = jnp.dot(xb, wr_ref[...], preferred_element_type=jnp.float32)

  return pl.pallas_call(
      body,
      grid=(n // blk,),
      in_specs=[
          pl.BlockSpec((blk, k), lambda i: (i, 0)),
          pl.BlockSpec((k, d), lambda i: (0, 0)),
          pl.BlockSpec((k, d), lambda i: (0, 0)),
      ],
      out_specs=[
          pl.BlockSpec((blk, d), lambda i: (i, 0)),
          pl.BlockSpec((blk, d), lambda i: (i, 0)),
      ],
      out_shape=[
          jax.ShapeDtypeStruct((n, d), jnp.float32),
          jax.ShapeDtypeStruct((n, d), jnp.float32),
      ],
  )(x, wl, wr)


def _mid_layer_call(agg, degp, r, b, wl, wr, blk, first):
  _, n, d = agg.shape
  do = wl.shape[1]

  def body(a_ref, g_ref, r_ref, b_ref, wl_ref, wr_ref, *o_refs):
    a = a_ref[0] + a_ref[1]
    if first:
      deg = jnp.maximum(g_ref[0] + g_ref[1], 1.0)
      dinv = 1.0 / deg
    else:
      dinv = g_ref[...]
    h = jnp.maximum(a * dinv[:, 0:1] + r_ref[...] + b_ref[...], 0.0)
    o_refs[0][...] = jnp.dot(h, wl_ref[...], preferred_element_type=jnp.float32)
    o_refs[1][...] = jnp.dot(h, wr_ref[...], preferred_element_type=jnp.float32)
    if first:
      o_refs[2][...] = dinv

  g_spec = (pl.BlockSpec((NC, blk, 16), lambda i: (0, i, 0)) if first
            else pl.BlockSpec((blk, 16), lambda i: (i, 0)))
  out_specs = [pl.BlockSpec((blk, do), lambda i: (i, 0)),
               pl.BlockSpec((blk, do), lambda i: (i, 0))]
  out_shape = [jax.ShapeDtypeStruct((n, do), jnp.float32),
               jax.ShapeDtypeStruct((n, do), jnp.float32)]
  if first:
    out_specs.append(pl.BlockSpec((blk, 16), lambda i: (i, 0)))
    out_shape.append(jax.ShapeDtypeStruct((n, 16), jnp.float32))

  return pl.pallas_call(
      body,
      grid=(n // blk,),
      in_specs=[
          pl.BlockSpec((NC, blk, d), lambda i: (0, i, 0)),
          g_spec,
          pl.BlockSpec((blk, d), lambda i: (i, 0)),
          pl.BlockSpec((1, d), lambda i: (0, 0)),
          pl.BlockSpec((d, do), lambda i: (0, 0)),
          pl.BlockSpec((d, do), lambda i: (0, 0)),
      ],
      out_specs=out_specs,
      out_shape=out_shape,
  )(agg, degp, r, b, wl, wr)


def _final_call(agg, dinv, r, b, blk):
  _, n, d = agg.shape

  def body(a_ref, g_ref, r_ref, b_ref, o_ref):
    a = a_ref[0] + a_ref[1]
    o_ref[...] = a * g_ref[...][:, 0:1] + r_ref[...] + b_ref[...]

  return pl.pallas_call(
      body,
      grid=(n // blk,),
      in_specs=[
          pl.BlockSpec((NC, blk, d), lambda i: (0, i, 0)),
          pl.BlockSpec((blk, 16), lambda i: (i, 0)),
          pl.BlockSpec((blk, d), lambda i: (i, 0)),
          pl.BlockSpec((1, d), lambda i: (0, 0)),
      ],
      out_specs=pl.BlockSpec((blk, d), lambda i: (i, 0)),
      out_shape=jax.ShapeDtypeStruct((n, d), jnp.float32),
  )(agg, dinv, r, b)


def kernel(x, edge_index, Wl1, Wr1, b1, Wl2, Wr2, b2, Wl3, Wr3, b3):
  n, d_in = x.shape
  e = edge_index.shape[1]
  d_h = Wl1.shape[1]
  n_cls = Wl3.shape[1]
  do = 16

  n_pad = -(-(n + 1) // 128) * 128
  per_tile = -(-e // (NW * 2 * CH)) * 2 * CH
  n_chunks = per_tile // CH
  e_pad = per_tile * NW

  src = edge_index[0]
  dst = edge_index[1]
  pad = e_pad - e
  pad_src = (jnp.arange(pad, dtype=jnp.int32) * 97) % n
  pad_dst = n + (jnp.arange(pad, dtype=jnp.int32) % (n_pad - n))
  srcC = jnp.concatenate([src, pad_src]).reshape(NW, n_chunks, CH)
  dstC = jnp.concatenate([dst, pad_dst]).reshape(NW, n_chunks, CH)

  zeros_d = jnp.zeros((n_pad, d_h), jnp.float32)
  zeros_16 = jnp.zeros((n_pad, 16), jnp.float32)
  ones_16 = jnp.ones((CH, 16), jnp.float32)

  wl3p = jnp.zeros((d_h, do), jnp.float32).at[:, :n_cls].set(Wl3)
  wr3p = jnp.zeros((d_h, do), jnp.float32).at[:, :n_cls].set(Wr3)
  b3p = jnp.zeros((1, do), jnp.float32).at[0, :n_cls].set(b3)

  blk = 1000

  p1, r1 = _proj2_call(x, Wl1, Wr1, blk)
  agg1, degp = _make_sc_agg(n_pad, d_h, n_chunks, True)(
      p1, srcC, dstC, zeros_d, zeros_16, ones_16)
  p2, r2, dinv = _mid_layer_call(
      agg1[:, :n], degp[:, :n], r1, b1.reshape(1, d_h), Wl2, Wr2, blk, True)

  agg2, = _make_sc_agg(n_pad, d_h, n_chunks, False)(p2, srcC, dstC, zeros_d)
  p3, r3 = _mid_layer_call(
      agg2[:, :n], dinv, r2, b2.reshape(1, d_h), wl3p, wr3p, blk, False)

  zeros_do = zeros_16 if do == 16 else jnp.zeros((n_pad, do), jnp.float32)
  agg3, = _make_sc_agg(n_pad, do, n_chunks, False)(p3, srcC, dstC, zeros_do)
  out = _final_call(agg3[:, :n], dinv, r3, b3p, blk)
  return out[:, :n_cls]
